# baseline (speedup 1.0000x reference)
import functools

import jax
import jax.numpy as jnp
from jax import lax
from jax.experimental import pallas as pl
from jax.experimental.pallas import tpu as pltpu
from jax.experimental.pallas import tpu_sc as plsc

DIM = 64
SCALE = 8.0

NC, NS = 2, 16
NW = NC * NS

CB = 256


def _unpack_table(tab_t, tail2, *, v):
    nfull = v // CB
    tailr = v - nfull * CB
    base_blocks = nfull // NW
    extra = nfull - base_blocks * NW

    mesh = plsc.VectorSubcoreMesh(core_axis_name="c", subcore_axis_name="s")

    @functools.partial(
        pl.kernel,
        out_type=jax.ShapeDtypeStruct((v // 2, 128), jnp.float32),
        mesh=mesh,
        scratch_types=[
            pltpu.VMEM((2, DIM, CB), jnp.float32),
            pltpu.VMEM((2, CB // 2, 128), jnp.float32),
            pltpu.VMEM((DIM, 128), jnp.float32),
            pltpu.SemaphoreType.DMA,
            pltpu.SemaphoreType.DMA,
            pltpu.SemaphoreType.DMA,
            pltpu.SemaphoreType.DMA,
        ],
        compiler_params=pltpu.CompilerParams(needs_layout_passes=False),
    )
    def k1(tab, tl, u, buf_a, buf_b, buf_t, g0, g1, o0, o1):
        gsem = (g0, g1)
        osem = (o0, o1)
        wid = lax.axis_index("s") * NC + lax.axis_index("c")

        def stage(slot, b):
            pltpu.async_copy(
                tab.at[:, pl.ds(b * CB, CB)], buf_a.at[slot], gsem[slot]
            )

        def drain_in(slot):
            pltpu.make_async_copy(
                tab.at[:, pl.ds(0, CB)], buf_a.at[slot], gsem[slot]
            ).wait()

        def drain_out(slot):
            pltpu.make_async_copy(
                u.at[pl.ds(0, CB // 2)], buf_b.at[slot], osem[slot]
            ).wait()

        iotas = [lax.iota(jnp.int32, 16) + (16 * c) for c in range(DIM // 16)]
        zeros = jnp.zeros((16,), jnp.int32)

        def emit(slot, rows, src):
            @plsc.parallel_loop(0, rows, unroll=4)
            def _(p):
                cols = [zeros + (2 * p + h) for h in range(2)]
                vals = [
                    plsc.load_gather(src, [iotas[c], cols[h]]) * SCALE
                    for h in range(2)
                    for c in range(DIM // 16)
                ]
                k = 0
                for h in range(2):
                    for c in range(DIM // 16):
                        buf_b[slot, p, pl.ds(h * DIM + c * 16, 16)] = vals[k]
                        k += 1

        def flush(slot, b):
            pltpu.async_copy(
                buf_b.at[slot],
                u.at[pl.ds(b * (CB // 2), CB // 2)],
                osem[slot],
            )

        def blk(j):
            return wid + NW * j

        stage(0, blk(0))
        stage(1, blk(1))

        @pl.loop(0, base_blocks // 2 - 1)
        def _(i):
            j0 = 2 * i
            drain_in(0)
            emit(0, CB // 2, buf_a.at[0])
            stage(0, blk(j0 + 2))
            flush(0, blk(j0))
            drain_in(1)
            emit(1, CB // 2, buf_a.at[1])
            stage(1, blk(j0 + 3))
            flush(1, blk(j0 + 1))
            drain_out(0)
            drain_out(1)

        drain_in(0)
        emit(0, CB // 2, buf_a.at[0])
        flush(0, blk(base_blocks - 2))
        drain_in(1)
        emit(1, CB // 2, buf_a.at[1])
        flush(1, blk(base_blocks - 1))
        drain_out(0)
        drain_out(1)

        @pl.when(wid < extra)
        def _():
            b = nfull - extra + wid
            stage(0, b)
            drain_in(0)
            emit(0, CB // 2, buf_a.at[0])
            flush(0, b)
            drain_out(0)

        if tailr:
            @pl.when(wid == extra)
            def _():
                pltpu.async_copy(tl, buf_t, g0)
                pltpu.make_async_copy(tl, buf_t, g0).wait()
                emit(0, tailr // 2, buf_t)
                pltpu.async_copy(
                    buf_b.at[0, pl.ds(0, tailr // 2)],
                    u.at[pl.ds(nfull * (CB // 2), tailr // 2)],
                    o0,
                )
                pltpu.make_async_copy(
                    u.at[pl.ds(0, tailr // 2)],
                    buf_b.at[0, pl.ds(0, tailr // 2)],
                    o0,
                ).wait()

    return k1(tab_t, tail2)


def _gather_emit(x5, u2, *, n_tb, n_c, v):
    units = 8 * n_tb

    mesh = plsc.VectorSubcoreMesh(core_axis_name="c", subcore_axis_name="s")

    @functools.partial(
        pl.kernel,
        out_type=jax.ShapeDtypeStruct((8 * n_tb, 8, n_c, 8, 128), jnp.float32),
        mesh=mesh,
        scratch_types=[
            pltpu.VMEM((n_tb, 8, 128), jnp.int32),
            pltpu.VMEM((2, 128, DIM), jnp.float32),
            pltpu.VMEM((2, 8, 8, 128), jnp.float32),
            pltpu.SemaphoreType.DMA,
            pltpu.SemaphoreType.DMA,
            pltpu.SemaphoreType.DMA,
            pltpu.SemaphoreType.DMA,
        ],
        compiler_params=pltpu.CompilerParams(
            use_tc_tiling_on_sc=False, needs_layout_passes=False
        ),
    )
    def k2(xr, u, out5, idx_v, g_v, o_v, g0, g1, o0, o1):
        gsem = (g0, g1)
        osem = (o0, o1)
        wid = lax.axis_index("s") * NC + lax.axis_index("c")

        pltpu.sync_copy(xr.at[:, wid], idx_v)

        def stage(slot, j):
            pltpu.async_copy(
                u.at[idx_v.at[j // 8, j % 8]], g_v.at[slot], gsem[slot]
            )

        def drain_g(slot):
            pltpu.make_async_copy(
                u.at[pl.ds(0, 128)], g_v.at[slot], gsem[slot]
            ).wait()

        def drain_o(slot):
            pltpu.make_async_copy(
                out5.at[0, :, 0], o_v.at[slot], osem[slot]
            ).wait()

        iotas = [lax.iota(jnp.int32, 16) + (16 * g) for g in range(8)]
        zeros = jnp.zeros((16,), jnp.int32)

        def emit(slot):
            @plsc.parallel_loop(0, DIM, unroll=4)
            def _(i):
                col = zeros + i
                vals = [
                    plsc.load_gather(g_v.at[slot], [iotas[g], col])
                    for g in range(8)
                ]
                for g in range(8):
                    o_v[slot, i // 8, i % 8, pl.ds(16 * g, 16)] = vals[g]

        def flush(slot, j):
            pltpu.async_copy(o_v.at[slot], out5.at[j, :, wid], osem[slot])

        stage(0, 0)
        stage(1, 1)

        @pl.loop(0, units // 2 - 1)
        def _(i):
            j0 = 2 * i
            drain_g(0)
            emit(0)
            stage(0, j0 + 2)
            flush(0, j0)
            drain_g(1)
            emit(1)
            stage(1, j0 + 3)
            flush(1, j0 + 1)
            drain_o(0)
            drain_o(1)

        drain_g(0)
        emit(0)
        flush(0, units - 2)
        drain_g(1)
        emit(1)
        flush(1, units - 1)
        drain_o(0)
        drain_o(1)

    return k2(x5, u2)


def kernel(x, table):
    b0, b1 = x.shape
    v = table.shape[0]
    n_c = b0 // 128
    n_tb = b1 // 8
    x5 = (
        x.astype(jnp.int32)
        .reshape(n_c, 128, n_tb, 8)
        .transpose(2, 0, 3, 1)
    )
    tailr = v % CB
    tail2 = jnp.pad(table[v - tailr:].T, ((0, 0), (0, 128 - tailr)))
    u = _unpack_table(table.T, tail2, v=v)
    u2 = u.reshape(v, DIM)
    out5 = _gather_emit(x5, u2, n_tb=n_tb, n_c=n_c, v=v)
    return out5.transpose(2, 4, 0, 1, 3).reshape(b0, b1, DIM)


# speedup vs baseline: 5.2624x; 5.2624x over previous
import functools

import jax
import jax.numpy as jnp
from jax import lax
from jax.experimental import pallas as pl
from jax.experimental.pallas import tpu as pltpu
from jax.experimental.pallas import tpu_sc as plsc

DIM = 64
SCALE = 8.0

NC, NS = 2, 16
NW = NC * NS

CB = 256


def _unpack_table(tab_t, tail2, *, v):
    nfull = v // CB
    tailr = v - nfull * CB
    base_blocks = nfull // NW
    extra = nfull - base_blocks * NW

    mesh = plsc.VectorSubcoreMesh(core_axis_name="c", subcore_axis_name="s")

    @functools.partial(
        pl.kernel,
        out_type=jax.ShapeDtypeStruct((v // 2, 128), jnp.float32),
        mesh=mesh,
        scratch_types=[
            pltpu.VMEM((2, DIM, CB), jnp.float32),
            pltpu.VMEM((2, CB // 2, 128), jnp.float32),
            pltpu.VMEM((DIM, 128), jnp.float32),
            pltpu.SemaphoreType.DMA,
            pltpu.SemaphoreType.DMA,
            pltpu.SemaphoreType.DMA,
            pltpu.SemaphoreType.DMA,
        ],
        compiler_params=pltpu.CompilerParams(needs_layout_passes=False),
    )
    def k1(tab, tl, u, buf_a, buf_b, buf_t, g0, g1, o0, o1):
        gsem = (g0, g1)
        osem = (o0, o1)
        wid = lax.axis_index("s") * NC + lax.axis_index("c")

        def stage(slot, b):
            pltpu.async_copy(
                tab.at[:, pl.ds(b * CB, CB)], buf_a.at[slot], gsem[slot]
            )

        def drain_in(slot):
            pltpu.make_async_copy(
                tab.at[:, pl.ds(0, CB)], buf_a.at[slot], gsem[slot]
            ).wait()

        def drain_out(slot):
            pltpu.make_async_copy(
                u.at[pl.ds(0, CB // 2)], buf_b.at[slot], osem[slot]
            ).wait()

        iotas = [lax.iota(jnp.int32, 16) + (16 * c) for c in range(DIM // 16)]
        zeros = jnp.zeros((16,), jnp.int32)

        iota16 = lax.iota(jnp.int32, 16)

        def emit(slot, cols, src):
            @plsc.parallel_loop(0, cols, unroll=2)
            def _(i):
                q_vec = (iota16 + i) & (cols - 1)
                p_vec = q_vec >> 1
                h64 = (q_vec & 1) << 6
                for c in range(DIM // 16):
                    vals = plsc.load_gather(src, [iotas[c], q_vec]) * SCALE
                    plsc.store_scatter(
                        buf_b.at[slot], [p_vec, h64 + iotas[c]], vals
                    )

        def flush(slot, b):
            pltpu.async_copy(
                buf_b.at[slot],
                u.at[pl.ds(b * (CB // 2), CB // 2)],
                osem[slot],
            )

        def blk(j):
            return wid + NW * j

        stage(0, blk(0))
        stage(1, blk(1))

        @pl.loop(0, base_blocks // 2 - 1)
        def _(i):
            j0 = 2 * i
            drain_in(0)
            emit(0, CB, buf_a.at[0])
            stage(0, blk(j0 + 2))
            flush(0, blk(j0))
            drain_in(1)
            emit(1, CB, buf_a.at[1])
            stage(1, blk(j0 + 3))
            flush(1, blk(j0 + 1))
            drain_out(0)
            drain_out(1)

        drain_in(0)
        emit(0, CB, buf_a.at[0])
        flush(0, blk(base_blocks - 2))
        drain_in(1)
        emit(1, CB, buf_a.at[1])
        flush(1, blk(base_blocks - 1))
        drain_out(0)
        drain_out(1)

        @pl.when(wid < extra)
        def _():
            b = nfull - extra + wid
            stage(0, b)
            drain_in(0)
            emit(0, CB, buf_a.at[0])
            flush(0, b)
            drain_out(0)

        if tailr:
            @pl.when(wid == extra)
            def _():
                pltpu.async_copy(tl, buf_t, g0)
                pltpu.make_async_copy(tl, buf_t, g0).wait()
                emit(0, tailr, buf_t)
                pltpu.async_copy(
                    buf_b.at[0, pl.ds(0, tailr // 2)],
                    u.at[pl.ds(nfull * (CB // 2), tailr // 2)],
                    o0,
                )
                pltpu.make_async_copy(
                    u.at[pl.ds(0, tailr // 2)],
                    buf_b.at[0, pl.ds(0, tailr // 2)],
                    o0,
                ).wait()

    return k1(tab_t, tail2)


def _gather_emit(x5, u2, *, n_tb, n_c, v):
    units = 8 * n_tb

    mesh = plsc.VectorSubcoreMesh(core_axis_name="c", subcore_axis_name="s")

    @functools.partial(
        pl.kernel,
        out_type=jax.ShapeDtypeStruct((8 * n_tb, 8, n_c, 8, 128), jnp.float32),
        mesh=mesh,
        scratch_types=[
            pltpu.VMEM((n_tb, 8, 128), jnp.int32),
            pltpu.VMEM((2, 128, DIM), jnp.float32),
            pltpu.VMEM((2, 8, 8, 128), jnp.float32),
            pltpu.SemaphoreType.DMA,
            pltpu.SemaphoreType.DMA,
            pltpu.SemaphoreType.DMA,
            pltpu.SemaphoreType.DMA,
        ],
        compiler_params=pltpu.CompilerParams(
            use_tc_tiling_on_sc=False, needs_layout_passes=False
        ),
    )
    def k2(xr, u, out5, idx_v, g_v, o_v, g0, g1, o0, o1):
        gsem = (g0, g1)
        osem = (o0, o1)
        wid = lax.axis_index("s") * NC + lax.axis_index("c")

        pltpu.sync_copy(xr.at[:, wid], idx_v)

        def stage(slot, j):
            pltpu.async_copy(
                u.at[idx_v.at[j // 8, j % 8]], g_v.at[slot], gsem[slot]
            )

        def drain_g(slot):
            pltpu.make_async_copy(
                u.at[pl.ds(0, 128)], g_v.at[slot], gsem[slot]
            ).wait()

        def drain_o(slot):
            pltpu.make_async_copy(
                out5.at[0, :, 0], o_v.at[slot], osem[slot]
            ).wait()

        iotas = [lax.iota(jnp.int32, 16) + (16 * g) for g in range(8)]
        zeros = jnp.zeros((16,), jnp.int32)

        iota16 = lax.iota(jnp.int32, 16)

        def emit(slot):
            @plsc.parallel_loop(0, DIM, unroll=2)
            def _(i):
                d_vec = (iota16 + i) & (DIM - 1)
                s_vec = d_vec >> 3
                r_vec = d_vec & 7
                for g in range(8):
                    vals = plsc.load_gather(g_v.at[slot], [iotas[g], d_vec])
                    plsc.store_scatter(
                        o_v.at[slot], [s_vec, r_vec, iotas[g]], vals
                    )

        def flush(slot, j):
            pltpu.async_copy(o_v.at[slot], out5.at[j, :, wid], osem[slot])

        stage(0, 0)
        stage(1, 1)

        @pl.loop(0, units // 2 - 1)
        def _(i):
            j0 = 2 * i
            drain_g(0)
            emit(0)
            stage(0, j0 + 2)
            flush(0, j0)
            drain_g(1)
            emit(1)
            stage(1, j0 + 3)
            flush(1, j0 + 1)
            drain_o(0)
            drain_o(1)

        drain_g(0)
        emit(0)
        flush(0, units - 2)
        drain_g(1)
        emit(1)
        flush(1, units - 1)
        drain_o(0)
        drain_o(1)

    return k2(x5, u2)


def kernel(x, table):
    b0, b1 = x.shape
    v = table.shape[0]
    n_c = b0 // 128
    n_tb = b1 // 8
    x5 = (
        x.astype(jnp.int32)
        .reshape(n_c, 128, n_tb, 8)
        .transpose(2, 0, 3, 1)
    )
    tailr = v % CB
    tail2 = jnp.pad(table[v - tailr:].T, ((0, 0), (0, 128 - tailr)))
    u = _unpack_table(table.T, tail2, v=v)
    u2 = u.reshape(v, DIM)
    out5 = _gather_emit(x5, u2, n_tb=n_tb, n_c=n_c, v=v)
    return out5.transpose(2, 4, 0, 1, 3).reshape(b0, b1, DIM)


# speedup vs baseline: 5.6516x; 1.0740x over previous
import functools

import jax
import jax.numpy as jnp
from jax import lax
from jax.experimental import pallas as pl
from jax.experimental.pallas import tpu as pltpu
from jax.experimental.pallas import tpu_sc as plsc

DIM = 64
SCALE = 8.0

NC, NS = 2, 16
NW = NC * NS

CB = 256


def _unpack_table(tab_t, tail2, *, v):
    nfull = v // CB
    tailr = v - nfull * CB
    base_blocks = nfull // NW
    extra = nfull - base_blocks * NW

    mesh = plsc.VectorSubcoreMesh(core_axis_name="c", subcore_axis_name="s")

    @functools.partial(
        pl.kernel,
        out_type=jax.ShapeDtypeStruct((v // 2, 128), jnp.float32),
        mesh=mesh,
        scratch_types=[
            pltpu.VMEM((2, DIM, CB), jnp.float32),
            pltpu.VMEM((2, CB // 2, 128), jnp.float32),
            pltpu.VMEM((DIM, 128), jnp.float32),
            pltpu.SemaphoreType.DMA,
            pltpu.SemaphoreType.DMA,
            pltpu.SemaphoreType.DMA,
            pltpu.SemaphoreType.DMA,
        ],
        compiler_params=pltpu.CompilerParams(needs_layout_passes=False),
    )
    def k1(tab, tl, u, buf_a, buf_b, buf_t, g0, g1, o0, o1):
        gsem = (g0, g1)
        osem = (o0, o1)
        wid = lax.axis_index("s") * NC + lax.axis_index("c")

        def stage(slot, b):
            pltpu.async_copy(
                tab.at[:, pl.ds(b * CB, CB)], buf_a.at[slot], gsem[slot]
            )

        def drain_in(slot):
            pltpu.make_async_copy(
                tab.at[:, pl.ds(0, CB)], buf_a.at[slot], gsem[slot]
            ).wait()

        def drain_out(slot):
            pltpu.make_async_copy(
                u.at[pl.ds(0, CB // 2)], buf_b.at[slot], osem[slot]
            ).wait()

        iotas = [lax.iota(jnp.int32, 16) + (16 * c) for c in range(DIM // 16)]
        zeros = jnp.zeros((16,), jnp.int32)

        iota16 = lax.iota(jnp.int32, 16)

        def emit(slot, cols, src):
            @plsc.parallel_loop(0, cols, unroll=2)
            def _(i):
                q_vec = (iota16 + i) & (cols - 1)
                p_vec = q_vec >> 1
                h64 = (q_vec & 1) << 6
                for c in range(DIM // 16):
                    vals = plsc.load_gather(src, [iotas[c], q_vec]) * SCALE
                    plsc.store_scatter(
                        buf_b.at[slot], [p_vec, h64 + iotas[c]], vals
                    )

        def flush(slot, b):
            pltpu.async_copy(
                buf_b.at[slot],
                u.at[pl.ds(b * (CB // 2), CB // 2)],
                osem[slot],
            )

        def blk(j):
            return wid + NW * j

        stage(0, blk(0))
        stage(1, blk(1))

        @pl.loop(0, base_blocks // 2 - 1)
        def _(i):
            j0 = 2 * i
            drain_in(0)
            emit(0, CB, buf_a.at[0])
            stage(0, blk(j0 + 2))
            flush(0, blk(j0))
            drain_in(1)
            emit(1, CB, buf_a.at[1])
            stage(1, blk(j0 + 3))
            flush(1, blk(j0 + 1))
            drain_out(0)
            drain_out(1)

        drain_in(0)
        emit(0, CB, buf_a.at[0])
        flush(0, blk(base_blocks - 2))
        drain_in(1)
        emit(1, CB, buf_a.at[1])
        flush(1, blk(base_blocks - 1))
        drain_out(0)
        drain_out(1)

        @pl.when(wid < extra)
        def _():
            b = nfull - extra + wid
            stage(0, b)
            drain_in(0)
            emit(0, CB, buf_a.at[0])
            flush(0, b)
            drain_out(0)

        if tailr:
            @pl.when(wid == extra)
            def _():
                pltpu.async_copy(tl, buf_t, g0)
                pltpu.make_async_copy(tl, buf_t, g0).wait()
                emit(0, tailr, buf_t)
                pltpu.async_copy(
                    buf_b.at[0, pl.ds(0, tailr // 2)],
                    u.at[pl.ds(nfull * (CB // 2), tailr // 2)],
                    o0,
                )
                pltpu.make_async_copy(
                    u.at[pl.ds(0, tailr // 2)],
                    buf_b.at[0, pl.ds(0, tailr // 2)],
                    o0,
                ).wait()

    return k1(tab_t, tail2)


def _gather_emit(x5, u2, *, n_tb, n_c, v):
    units = 8 * n_tb

    mesh = plsc.VectorSubcoreMesh(core_axis_name="c", subcore_axis_name="s")

    @functools.partial(
        pl.kernel,
        out_type=jax.ShapeDtypeStruct((8 * n_tb, 8, n_c, 8, 128), jnp.float32),
        mesh=mesh,
        scratch_types=[
            pltpu.VMEM((n_tb, 8, 128), jnp.int32),
            pltpu.VMEM((4, 128, DIM), jnp.float32),
            pltpu.VMEM((4, 8, 8, 128), jnp.float32),
            pltpu.SemaphoreType.DMA,
            pltpu.SemaphoreType.DMA,
            pltpu.SemaphoreType.DMA,
            pltpu.SemaphoreType.DMA,
            pltpu.SemaphoreType.DMA,
            pltpu.SemaphoreType.DMA,
            pltpu.SemaphoreType.DMA,
            pltpu.SemaphoreType.DMA,
        ],
        compiler_params=pltpu.CompilerParams(
            use_tc_tiling_on_sc=False, needs_layout_passes=False
        ),
    )
    def k2(xr, u, out5, idx_v, g_v, o_v, g0, g1, g2, g3, o0, o1, o2, o3):
        gsem = (g0, g1, g2, g3)
        osem = (o0, o1, o2, o3)
        wid = lax.axis_index("s") * NC + lax.axis_index("c")

        pltpu.sync_copy(xr.at[:, wid], idx_v)

        def stage(slot, j):
            pltpu.async_copy(
                u.at[idx_v.at[j // 8, j % 8]], g_v.at[slot], gsem[slot]
            )

        def drain_g(slot):
            pltpu.make_async_copy(
                u.at[pl.ds(0, 128)], g_v.at[slot], gsem[slot]
            ).wait()

        def drain_o(slot):
            pltpu.make_async_copy(
                out5.at[0, :, 0], o_v.at[slot], osem[slot]
            ).wait()

        iotas = [lax.iota(jnp.int32, 16) + (16 * g) for g in range(8)]
        zeros = jnp.zeros((16,), jnp.int32)

        iota16 = lax.iota(jnp.int32, 16)

        def emit(slot):
            @plsc.parallel_loop(0, DIM, unroll=2)
            def _(i):
                d_vec = (iota16 + i) & (DIM - 1)
                s_vec = d_vec >> 3
                r_vec = d_vec & 7
                for g in range(8):
                    vals = plsc.load_gather(g_v.at[slot], [iotas[g], d_vec])
                    plsc.store_scatter(
                        o_v.at[slot], [s_vec, r_vec, iotas[g]], vals
                    )

        def flush(slot, j):
            pltpu.async_copy(o_v.at[slot], out5.at[j, :, wid], osem[slot])

        for t in range(4):
            stage(t, t)
        for t in range(4):
            drain_g(t)
            emit(t)
            stage(t, 4 + t)
            flush(t, t)

        @pl.loop(1, units // 4 - 1)
        def _(i):
            j0 = 4 * i
            for t in range(4):
                drain_g(t)
                drain_o(t)
                emit(t)
                stage(t, j0 + t + 4)
                flush(t, j0 + t)

        for t in range(4):
            drain_g(t)
            drain_o(t)
            emit(t)
            flush(t, units - 4 + t)
        for t in range(4):
            drain_o(t)

    return k2(x5, u2)


def kernel(x, table):
    b0, b1 = x.shape
    v = table.shape[0]
    n_c = b0 // 128
    n_tb = b1 // 8
    x5 = (
        x.astype(jnp.int32)
        .reshape(n_c, 128, n_tb, 8)
        .transpose(2, 0, 3, 1)
    )
    tailr = v % CB
    tail2 = jnp.pad(table[v - tailr:].T, ((0, 0), (0, 128 - tailr)))
    u = _unpack_table(table.T, tail2, v=v)
    u2 = u.reshape(v, DIM)
    out5 = _gather_emit(x5, u2, n_tb=n_tb, n_c=n_c, v=v)
    return out5.transpose(2, 4, 0, 1, 3).reshape(b0, b1, DIM)


# speedup vs baseline: 5.6530x; 1.0002x over previous
import functools

import jax
import jax.numpy as jnp
from jax import lax
from jax.experimental import pallas as pl
from jax.experimental.pallas import tpu as pltpu
from jax.experimental.pallas import tpu_sc as plsc

DIM = 64
SCALE = 8.0

NC, NS = 2, 16
NW = NC * NS

CB = 256


def _unpack_table(tab_t, tail2, *, v):
    nfull = v // CB
    tailr = v - nfull * CB
    base_blocks = nfull // NW
    extra = nfull - base_blocks * NW

    mesh = plsc.VectorSubcoreMesh(core_axis_name="c", subcore_axis_name="s")

    @functools.partial(
        pl.kernel,
        out_type=jax.ShapeDtypeStruct((v // 2, 128), jnp.float32),
        mesh=mesh,
        scratch_types=[
            pltpu.VMEM((2, DIM, CB), jnp.float32),
            pltpu.VMEM((2, CB // 2, 128), jnp.float32),
            pltpu.VMEM((DIM, 128), jnp.float32),
            pltpu.SemaphoreType.DMA,
            pltpu.SemaphoreType.DMA,
            pltpu.SemaphoreType.DMA,
            pltpu.SemaphoreType.DMA,
        ],
        compiler_params=pltpu.CompilerParams(needs_layout_passes=False),
    )
    def k1(tab, tl, u, buf_a, buf_b, buf_t, g0, g1, o0, o1):
        gsem = (g0, g1)
        osem = (o0, o1)
        wid = lax.axis_index("s") * NC + lax.axis_index("c")

        def stage(slot, b):
            pltpu.async_copy(
                tab.at[:, pl.ds(b * CB, CB)], buf_a.at[slot], gsem[slot]
            )

        def drain_in(slot):
            pltpu.make_async_copy(
                tab.at[:, pl.ds(0, CB)], buf_a.at[slot], gsem[slot]
            ).wait()

        def drain_out(slot):
            pltpu.make_async_copy(
                u.at[pl.ds(0, CB // 2)], buf_b.at[slot], osem[slot]
            ).wait()

        iotas = [lax.iota(jnp.int32, 16) + (16 * c) for c in range(DIM // 16)]
        zeros = jnp.zeros((16,), jnp.int32)

        iota16 = lax.iota(jnp.int32, 16)

        def emit(slot, cols, src):
            @plsc.parallel_loop(0, cols, unroll=2)
            def _(i):
                q_vec = (iota16 + i) & (cols - 1)
                p_vec = q_vec >> 1
                h64 = (q_vec & 1) << 6
                for c in range(DIM // 16):
                    vals = plsc.load_gather(src, [iotas[c], q_vec]) * SCALE
                    plsc.store_scatter(
                        buf_b.at[slot], [p_vec, h64 + iotas[c]], vals
                    )

        def flush(slot, b):
            pltpu.async_copy(
                buf_b.at[slot],
                u.at[pl.ds(b * (CB // 2), CB // 2)],
                osem[slot],
            )

        def blk(j):
            return wid + NW * j

        stage(0, blk(0))
        stage(1, blk(1))

        drain_in(0)
        emit(0, CB, buf_a.at[0])
        stage(0, blk(2))
        flush(0, blk(0))
        drain_in(1)
        emit(1, CB, buf_a.at[1])
        stage(1, blk(3))
        flush(1, blk(1))

        @pl.loop(1, base_blocks // 2 - 1)
        def _(i):
            j0 = 2 * i
            drain_in(0)
            drain_out(0)
            emit(0, CB, buf_a.at[0])
            stage(0, blk(j0 + 2))
            flush(0, blk(j0))
            drain_in(1)
            drain_out(1)
            emit(1, CB, buf_a.at[1])
            stage(1, blk(j0 + 3))
            flush(1, blk(j0 + 1))

        drain_in(0)
        drain_out(0)
        emit(0, CB, buf_a.at[0])
        flush(0, blk(base_blocks - 2))
        drain_in(1)
        drain_out(1)
        emit(1, CB, buf_a.at[1])
        flush(1, blk(base_blocks - 1))
        drain_out(0)
        drain_out(1)

        @pl.when(wid < extra)
        def _():
            b = nfull - extra + wid
            stage(0, b)
            drain_in(0)
            emit(0, CB, buf_a.at[0])
            flush(0, b)
            drain_out(0)

        if tailr:
            @pl.when(wid == extra)
            def _():
                pltpu.async_copy(tl, buf_t, g0)
                pltpu.make_async_copy(tl, buf_t, g0).wait()
                emit(0, tailr, buf_t)
                pltpu.async_copy(
                    buf_b.at[0, pl.ds(0, tailr // 2)],
                    u.at[pl.ds(nfull * (CB // 2), tailr // 2)],
                    o0,
                )
                pltpu.make_async_copy(
                    u.at[pl.ds(0, tailr // 2)],
                    buf_b.at[0, pl.ds(0, tailr // 2)],
                    o0,
                ).wait()

    return k1(tab_t, tail2)


def _gather_emit(x5, u2, *, n_tb, n_c, v):
    units = 8 * n_tb

    mesh = plsc.VectorSubcoreMesh(core_axis_name="c", subcore_axis_name="s")

    @functools.partial(
        pl.kernel,
        out_type=jax.ShapeDtypeStruct((8 * n_tb, 8, n_c, 8, 128), jnp.float32),
        mesh=mesh,
        scratch_types=[
            pltpu.VMEM((n_tb, 8, 128), jnp.int32),
            pltpu.VMEM((4, 128, DIM), jnp.float32),
            pltpu.VMEM((4, 8, 8, 128), jnp.float32),
            pltpu.SemaphoreType.DMA,
            pltpu.SemaphoreType.DMA,
            pltpu.SemaphoreType.DMA,
            pltpu.SemaphoreType.DMA,
            pltpu.SemaphoreType.DMA,
            pltpu.SemaphoreType.DMA,
            pltpu.SemaphoreType.DMA,
            pltpu.SemaphoreType.DMA,
        ],
        compiler_params=pltpu.CompilerParams(
            use_tc_tiling_on_sc=False, needs_layout_passes=False
        ),
    )
    def k2(xr, u, out5, idx_v, g_v, o_v, g0, g1, g2, g3, o0, o1, o2, o3):
        gsem = (g0, g1, g2, g3)
        osem = (o0, o1, o2, o3)
        wid = lax.axis_index("s") * NC + lax.axis_index("c")

        pltpu.sync_copy(xr.at[:, wid], idx_v)

        def stage(slot, j):
            pltpu.async_copy(
                u.at[idx_v.at[j // 8, j % 8]], g_v.at[slot], gsem[slot]
            )

        def drain_g(slot):
            pltpu.make_async_copy(
                u.at[pl.ds(0, 128)], g_v.at[slot], gsem[slot]
            ).wait()

        def drain_o(slot):
            pltpu.make_async_copy(
                out5.at[0, :, 0], o_v.at[slot], osem[slot]
            ).wait()

        iotas = [lax.iota(jnp.int32, 16) + (16 * g) for g in range(8)]
        zeros = jnp.zeros((16,), jnp.int32)

        iota16 = lax.iota(jnp.int32, 16)

        def emit(slot):
            @plsc.parallel_loop(0, DIM, unroll=2)
            def _(i):
                d_vec = (iota16 + i) & (DIM - 1)
                s_vec = d_vec >> 3
                r_vec = d_vec & 7
                for g in range(8):
                    vals = plsc.load_gather(g_v.at[slot], [iotas[g], d_vec])
                    plsc.store_scatter(
                        o_v.at[slot], [s_vec, r_vec, iotas[g]], vals
                    )

        def flush(slot, j):
            pltpu.async_copy(o_v.at[slot], out5.at[j, :, wid], osem[slot])

        for t in range(4):
            stage(t, t)
        for t in range(4):
            drain_g(t)
            emit(t)
            stage(t, 4 + t)
            flush(t, t)

        @pl.loop(1, units // 4 - 1)
        def _(i):
            j0 = 4 * i
            for t in range(4):
                drain_g(t)
                drain_o(t)
                emit(t)
                stage(t, j0 + t + 4)
                flush(t, j0 + t)

        for t in range(4):
            drain_g(t)
            drain_o(t)
            emit(t)
            flush(t, units - 4 + t)
        for t in range(4):
            drain_o(t)

    return k2(x5, u2)


def kernel(x, table):
    b0, b1 = x.shape
    v = table.shape[0]
    n_c = b0 // 128
    n_tb = b1 // 8
    x5 = (
        x.astype(jnp.int32)
        .reshape(n_c, 128, n_tb, 8)
        .transpose(2, 0, 3, 1)
    )
    tailr = v % CB
    tail2 = jnp.pad(table[v - tailr:].T, ((0, 0), (0, 128 - tailr)))
    u = _unpack_table(table.T, tail2, v=v)
    u2 = u.reshape(v, DIM)
    out5 = _gather_emit(x5, u2, n_tb=n_tb, n_c=n_c, v=v)
    return out5.transpose(2, 4, 0, 1, 3).reshape(b0, b1, DIM)
